# baseline (speedup 1.0000x reference)
import jax
import jax.numpy as jnp
from jax import lax
from jax.experimental import pallas as pl
from jax.experimental.pallas import tpu as pltpu
from jax.experimental.pallas import tpu_sc as plsc

N_EXPERTS = 64
TOPK = 8
N_GROUPS = 8
GROUP_SIZE = N_EXPERTS // N_GROUPS
TOPK_GROUPS = 4
ROUTE_SCALE = 2.5
BT = 512
NW = 32
LANES = 16
CHUNK = 128
NSLICES = 2


def _scores_body(x_ref, w_ref, bias_ref, s_out, sbt_out):
    logits = lax.dot_general(
        x_ref[...], w_ref[...], (((1,), (1,)), ((), ())),
        preferred_element_type=jnp.float32)
    s = jax.nn.sigmoid(logits)
    s_out[...] = s
    sbt = (s + bias_ref[...]).T
    for j in range(BT // CHUNK):
        sbt_out[j] = sbt[:, j * CHUNK:(j + 1) * CHUNK]


def _tc_scores(x, weight, bias_row, i0, nb):
    D = x.shape[1]
    return pl.pallas_call(
        _scores_body,
        grid=(nb,),
        in_specs=[
            pl.BlockSpec((BT, D), lambda i: (i0 + i, 0)),
            pl.BlockSpec((N_EXPERTS, D), lambda i: (0, 0)),
            pl.BlockSpec((1, N_EXPERTS), lambda i: (0, 0)),
        ],
        out_specs=[
            pl.BlockSpec((BT, N_EXPERTS), lambda i: (i, 0)),
            pl.BlockSpec((BT // CHUNK, N_EXPERTS, CHUNK), lambda i: (i, 0, 0)),
        ],
        out_shape=[
            jax.ShapeDtypeStruct((nb * BT, N_EXPERTS), jnp.float32),
            jax.ShapeDtypeStruct((nb * BT // CHUNK, N_EXPERTS, CHUNK),
                                 jnp.float32),
        ],
    )(x, weight, bias_row)


def _route_body(sbt_hbm, bias_hbm, w_hbm, i_hbm,
                buf_v, fm_v, bias_v, wout_v, iout_v, sem0, sem1):
    tpw = wout_v.shape[1]
    nck = tpw // CHUNK
    wid = lax.axis_index("s") * 2 + lax.axis_index("c")
    base = wid * tpw
    pltpu.sync_copy(bias_hbm, bias_v)

    iota = lax.broadcasted_iota(jnp.int32, (LANES,), 0)
    neg_inf = jnp.full((LANES,), -jnp.inf, jnp.float32)
    ones = jnp.full((LANES,), 1, jnp.int32)
    zeros = jnp.zeros((LANES,), jnp.int32)

    def vec(v):
        return jnp.full((LANES,), v, jnp.int32)

    def tree(pairs):
        while len(pairs) > 1:
            nxt = []
            for p in range(0, len(pairs), 2):
                (av, ai), (bv, bi) = pairs[p], pairs[p + 1]
                gt = bv > av
                nxt.append((jnp.where(gt, bv, av), jnp.where(gt, bi, ai)))
            pairs = nxt
        return pairs[0]

    def step(buf, ck, tg, carry):
        s0 = tg * LANES
        t0 = ck * CHUNK + s0
        gs, gi = [], []
        for g in range(N_GROUPS):
            wv, wi = tree([(buf[g * GROUP_SIZE + r, pl.ds(s0, LANES)],
                            vec(g * GROUP_SIZE + r))
                           for r in range(GROUP_SIZE)])
            gs.append(wv)
            gi.append(wi)
        keep = []
        for g in range(N_GROUPS):
            cnt = zeros
            for j in range(N_GROUPS):
                if j == g:
                    continue
                beats = (gs[j] >= gs[g]) if j < g else (gs[j] > gs[g])
                cnt = cnt + jnp.where(beats, ones, zeros)
            keep.append(cnt < vec(TOPK_GROUPS))
        for e in range(N_EXPERTS):
            v = buf[e, pl.ds(s0, LANES)]
            fm_v[pl.ds(e * LANES, LANES)] = jnp.where(
                keep[e // GROUP_SIZE], v, neg_inf)
        wvs = [jnp.where(keep[g], gs[g], neg_inf) for g in range(N_GROUPS)]
        wis = list(gi)
        for k in range(TOPK):
            best, bidx = tree(list(zip(wvs, wis)))
            kill = bidx * vec(LANES) + iota
            plsc.store_scatter(fm_v, [kill], neg_inf)
            bsel = plsc.load_gather(bias_v, [bidx])
            wv = (best - bsel) * jnp.full((LANES,), ROUTE_SCALE, jnp.float32)
            wout_v[k, pl.ds(t0, LANES)] = wv
            iout_v[k, pl.ds(t0, LANES)] = bidx
            if k == TOPK - 1:
                break
            bg128 = jnp.bitwise_and(bidx, vec(~(GROUP_SIZE - 1))) * vec(LANES)
            nv, ni = tree([
                (plsc.load_gather(
                    fm_v, [bg128 + vec(r * LANES) + iota]),
                 jnp.right_shift(bg128, vec(4)) + vec(r))
                for r in range(GROUP_SIZE)])
            bg = jnp.right_shift(bidx, vec(3))
            for g in range(N_GROUPS):
                hit = bg == vec(g)
                wvs[g] = jnp.where(hit, nv, wvs[g])
                wis[g] = jnp.where(hit, ni, wis[g])
        return carry

    sems = (sem0, sem1)
    if nck % 2 != 0:
        def chunk_loop(ck, carry):
            pltpu.sync_copy(sbt_hbm.at[wid * nck + ck], buf_v.at[0])
            return lax.fori_loop(
                0, CHUNK // LANES,
                lambda tg, c: step(buf_v.at[0], ck, tg, c), carry)

        lax.fori_loop(0, nck, chunk_loop, 0)
        pltpu.sync_copy(wout_v, w_hbm.at[:, pl.ds(base, tpw)])
        pltpu.sync_copy(iout_v, i_hbm.at[:, pl.ds(base, tpw)])
        return

    for b in range(2):
        pltpu.async_copy(sbt_hbm.at[wid * nck + b], buf_v.at[b], sems[b])

    def chunk_pair(cc, carry):
        c = carry
        for b in range(2):
            ck = cc * 2 + b
            pltpu.make_async_copy(
                sbt_hbm.at[wid * nck], buf_v.at[b], sems[b]).wait()

            @pl.when(ck + 2 < nck)
            def _():
                pltpu.async_copy(
                    sbt_hbm.at[wid * nck + ck + 2], buf_v.at[b], sems[b])

            c = lax.fori_loop(
                0, CHUNK // LANES,
                lambda tg, cy, _b=b, _ck=ck: step(buf_v.at[_b], _ck, tg, cy),
                c)
        return c

    lax.fori_loop(0, nck // 2, chunk_pair, 0)
    pltpu.sync_copy(wout_v, w_hbm.at[:, pl.ds(base, tpw)])
    pltpu.sync_copy(iout_v, i_hbm.at[:, pl.ds(base, tpw)])


def _sc_route(sbt, bias, sb_tokens):
    tpw = sb_tokens // NW
    mesh = plsc.VectorSubcoreMesh(core_axis_name="c", subcore_axis_name="s")
    f = pl.kernel(
        _route_body,
        out_type=[
            jax.ShapeDtypeStruct((TOPK, sb_tokens), jnp.float32),
            jax.ShapeDtypeStruct((TOPK, sb_tokens), jnp.int32),
        ],
        mesh=mesh,
        compiler_params=pltpu.CompilerParams(needs_layout_passes=False),
        scratch_types=[
            pltpu.VMEM((2, N_EXPERTS, CHUNK), jnp.float32),
            pltpu.VMEM((N_EXPERTS * LANES,), jnp.float32),
            pltpu.VMEM((N_EXPERTS,), jnp.float32),
            pltpu.VMEM((TOPK, tpw), jnp.float32),
            pltpu.VMEM((TOPK, tpw), jnp.int32),
            pltpu.SemaphoreType.DMA,
            pltpu.SemaphoreType.DMA,
        ],
    )
    return f(sbt, bias)


def kernel(x, weight, expert_bias):
    B, D = x.shape
    bias_row = expert_bias.reshape(1, N_EXPERTS)
    nb_all = B // BT
    slice_blocks = [nb_all - nb_all // 4, nb_all // 4]
    raws, wts, its = [], [], []
    i0 = 0
    for nb in slice_blocks:
        raw, sbt = _tc_scores(x, weight, bias_row, i0, nb)
        w_t, i_t = _sc_route(sbt, expert_bias, nb * BT)
        raws.append(raw)
        wts.append(w_t)
        its.append(i_t)
        i0 += nb
    raw = jnp.concatenate(raws, axis=0)
    w_t = jnp.concatenate(wts, axis=1)
    i_t = jnp.concatenate(its, axis=1)
    return w_t.T, i_t.T, raw


# speedup vs baseline: 1.0438x; 1.0438x over previous
import jax
import jax.numpy as jnp
from jax import lax
from jax.experimental import pallas as pl
from jax.experimental.pallas import tpu as pltpu
from jax.experimental.pallas import tpu_sc as plsc

N_EXPERTS = 64
TOPK = 8
N_GROUPS = 8
GROUP_SIZE = N_EXPERTS // N_GROUPS
TOPK_GROUPS = 4
ROUTE_SCALE = 2.5
BT = 512
NW = 32
LANES = 16
CHUNK = 128
NSLICES = 2


def _scores_body(x_ref, w_ref, bias_ref, s_out, sbt_out):
    logits = lax.dot_general(
        x_ref[...], w_ref[...], (((1,), (1,)), ((), ())),
        preferred_element_type=jnp.float32)
    s = jax.nn.sigmoid(logits)
    s_out[...] = s
    sbt = (s + bias_ref[...]).T
    for j in range(BT // CHUNK):
        sbt_out[j] = sbt[:, j * CHUNK:(j + 1) * CHUNK]


def _tc_scores(x, weight, bias_row, i0, nb):
    D = x.shape[1]
    return pl.pallas_call(
        _scores_body,
        grid=(nb,),
        in_specs=[
            pl.BlockSpec((BT, D), lambda i: (i0 + i, 0)),
            pl.BlockSpec((N_EXPERTS, D), lambda i: (0, 0)),
            pl.BlockSpec((1, N_EXPERTS), lambda i: (0, 0)),
        ],
        out_specs=[
            pl.BlockSpec((BT, N_EXPERTS), lambda i: (i, 0)),
            pl.BlockSpec((BT // CHUNK, N_EXPERTS, CHUNK), lambda i: (i, 0, 0)),
        ],
        out_shape=[
            jax.ShapeDtypeStruct((nb * BT, N_EXPERTS), jnp.float32),
            jax.ShapeDtypeStruct((nb * BT // CHUNK, N_EXPERTS, CHUNK),
                                 jnp.float32),
        ],
    )(x, weight, bias_row)


def _route_body(sbt_hbm, bias_hbm, w_hbm, i_hbm,
                buf_v, fm_v, bias_v, wout_v, iout_v, sem0, sem1):
    tpw = wout_v.shape[1]
    nck = tpw // CHUNK
    wid = lax.axis_index("s") * 2 + lax.axis_index("c")
    base = wid * tpw
    pltpu.sync_copy(bias_hbm, bias_v)

    iota = lax.broadcasted_iota(jnp.int32, (LANES,), 0)
    neg_inf = jnp.full((LANES,), -jnp.inf, jnp.float32)
    ones = jnp.full((LANES,), 1, jnp.int32)
    zeros = jnp.zeros((LANES,), jnp.int32)

    def vec(v):
        return jnp.full((LANES,), v, jnp.int32)

    def tree(pairs):
        while len(pairs) > 1:
            nxt = []
            for p in range(0, len(pairs), 2):
                (av, ai), (bv, bi) = pairs[p], pairs[p + 1]
                gt = bv > av
                nxt.append((jnp.where(gt, bv, av), jnp.where(gt, bi, ai)))
            pairs = nxt
        return pairs[0]

    def step(buf, ck, tg, carry):
        s0 = tg * LANES
        t0 = ck * CHUNK + s0
        gs, gi = [], []
        for g in range(N_GROUPS):
            wv, wi = tree([(buf[g * GROUP_SIZE + r, pl.ds(s0, LANES)],
                            vec(g * GROUP_SIZE + r))
                           for r in range(GROUP_SIZE)])
            gs.append(wv)
            gi.append(wi)
        keep = []
        for g in range(N_GROUPS):
            cnt = zeros
            for j in range(N_GROUPS):
                if j == g:
                    continue
                beats = (gs[j] >= gs[g]) if j < g else (gs[j] > gs[g])
                cnt = cnt + jnp.where(beats, ones, zeros)
            keep.append(cnt < vec(TOPK_GROUPS))
        for e in range(N_EXPERTS):
            v = buf[e, pl.ds(s0, LANES)]
            fm_v[pl.ds(e * LANES, LANES)] = jnp.where(
                keep[e // GROUP_SIZE], v, neg_inf)
        wvs = [jnp.where(keep[g], gs[g], neg_inf) for g in range(N_GROUPS)]
        wis = list(gi)
        for k in range(TOPK):
            best, bidx = tree(list(zip(wvs, wis)))
            kill = bidx * vec(LANES) + iota
            plsc.store_scatter(fm_v, [kill], neg_inf)
            bsel = plsc.load_gather(bias_v, [bidx])
            wv = (best - bsel) * jnp.full((LANES,), ROUTE_SCALE, jnp.float32)
            wout_v[k, pl.ds(t0, LANES)] = wv
            iout_v[k, pl.ds(t0, LANES)] = bidx
            if k == TOPK - 1:
                break
            bg128 = jnp.bitwise_and(bidx, vec(~(GROUP_SIZE - 1))) * vec(LANES)
            nv, ni = tree([
                (plsc.load_gather(
                    fm_v, [bg128 + vec(r * LANES) + iota]),
                 jnp.right_shift(bg128, vec(4)) + vec(r))
                for r in range(GROUP_SIZE)])
            bg = jnp.right_shift(bidx, vec(3))
            for g in range(N_GROUPS):
                hit = bg == vec(g)
                wvs[g] = jnp.where(hit, nv, wvs[g])
                wis[g] = jnp.where(hit, ni, wis[g])
        return carry

    sems = (sem0, sem1)
    if nck % 2 != 0:
        def chunk_loop(ck, carry):
            pltpu.sync_copy(sbt_hbm.at[wid * nck + ck], buf_v.at[0])
            return lax.fori_loop(
                0, CHUNK // LANES,
                lambda tg, c: step(buf_v.at[0], ck, tg, c), carry)

        lax.fori_loop(0, nck, chunk_loop, 0)
        pltpu.sync_copy(wout_v, w_hbm.at[:, pl.ds(base, tpw)])
        pltpu.sync_copy(iout_v, i_hbm.at[:, pl.ds(base, tpw)])
        return

    for b in range(2):
        pltpu.async_copy(sbt_hbm.at[wid * nck + b], buf_v.at[b], sems[b])

    def chunk_pair(cc, carry):
        c = carry
        for b in range(2):
            ck = cc * 2 + b
            pltpu.make_async_copy(
                sbt_hbm.at[wid * nck], buf_v.at[b], sems[b]).wait()

            @pl.when(ck + 2 < nck)
            def _():
                pltpu.async_copy(
                    sbt_hbm.at[wid * nck + ck + 2], buf_v.at[b], sems[b])

            c = lax.fori_loop(
                0, CHUNK // LANES,
                lambda tg, cy, _b=b, _ck=ck: step(buf_v.at[_b], _ck, tg, cy),
                c)
        return c

    lax.fori_loop(0, nck // 2, chunk_pair, 0)
    pltpu.sync_copy(wout_v, w_hbm.at[:, pl.ds(base, tpw)])
    pltpu.sync_copy(iout_v, i_hbm.at[:, pl.ds(base, tpw)])


def _sc_route(sbt, bias, sb_tokens):
    tpw = sb_tokens // NW
    mesh = plsc.VectorSubcoreMesh(core_axis_name="c", subcore_axis_name="s")
    f = pl.kernel(
        _route_body,
        out_type=[
            jax.ShapeDtypeStruct((TOPK, sb_tokens), jnp.float32),
            jax.ShapeDtypeStruct((TOPK, sb_tokens), jnp.int32),
        ],
        mesh=mesh,
        compiler_params=pltpu.CompilerParams(needs_layout_passes=False),
        scratch_types=[
            pltpu.VMEM((2, N_EXPERTS, CHUNK), jnp.float32),
            pltpu.VMEM((N_EXPERTS * LANES,), jnp.float32),
            pltpu.VMEM((N_EXPERTS,), jnp.float32),
            pltpu.VMEM((TOPK, tpw), jnp.float32),
            pltpu.VMEM((TOPK, tpw), jnp.int32),
            pltpu.SemaphoreType.DMA,
            pltpu.SemaphoreType.DMA,
        ],
    )
    return f(sbt, bias)


def kernel(x, weight, expert_bias):
    B, D = x.shape
    bias_row = expert_bias.reshape(1, N_EXPERTS)
    nb_all = B // BT
    slice_blocks = [nb_all // 2, nb_all - nb_all // 2]
    raws, wts, its = [], [], []
    i0 = 0
    for nb in slice_blocks:
        raw, sbt = _tc_scores(x, weight, bias_row, i0, nb)
        w_t, i_t = _sc_route(sbt, expert_bias, nb * BT)
        raws.append(raw)
        wts.append(w_t)
        its.append(i_t)
        i0 += nb
    raw = jnp.concatenate(raws, axis=0)
    w_t = jnp.concatenate(wts, axis=1)
    i_t = jnp.concatenate(its, axis=1)
    return w_t.T, i_t.T, raw
